# baseline (speedup 1.0000x reference)
import functools

import jax
import jax.numpy as jnp
from jax import lax
from jax.experimental import pallas as pl
from jax.experimental.pallas import tpu as pltpu
from jax.experimental.pallas import tpu_sc as plsc

VOCAB_SIZE = 10000
BATCH_SIZE = 4096
PRIOR_VAL = 1.0

NUM_CORES = 2
NUM_SUBCORES = 16
LANES = 16
NUM_WORKERS = NUM_CORES * NUM_SUBCORES
ROWS_PER_WORKER = BATCH_SIZE // NUM_WORKERS
CHUNK = 8
NUM_CHUNKS = ROWS_PER_WORKER // CHUNK
VECS_PER_ROW = VOCAB_SIZE // LANES

LN2 = 0.6931471805599453
SQRT2 = 1.4142135623730951
_LOG_COEFFS = (
    3.3423269089893903e-08,
    1.000003098647089,
    -0.5000129330593959,
    0.33304812395033884,
    -0.24911210645380452,
    0.2061178523941565,
    -0.18627697325890152,
    0.11448435453731831,
)


def _ln_vec(x):
    bits = plsc.bitcast(x, jnp.int32)
    e = (bits >> 23) - 127
    m = plsc.bitcast(
        (bits & jnp.int32(0x007FFFFF)) | jnp.int32(0x3F800000), jnp.float32
    )
    big = m > jnp.float32(SQRT2)
    m = jnp.where(big, m * jnp.float32(0.5), m)
    e = jnp.where(big, e + 1, e)
    t = m - jnp.float32(1.0)
    p = jnp.full((LANES,), jnp.float32(_LOG_COEFFS[7]))
    for c in _LOG_COEFFS[6::-1]:
        p = p * t + jnp.float32(c)
    return e.astype(jnp.float32) * jnp.float32(LN2) + p


def _sc_body(table_hbm, idx_hbm, out_hbm, idx_v, rows_2d, sem):
    wid = lax.axis_index("s") * NUM_CORES + lax.axis_index("c")
    base = wid * ROWS_PER_WORKER
    pltpu.sync_copy(idx_hbm.at[pl.ds(base, ROWS_PER_WORKER)], idx_v)

    lane_iota = lax.iota(jnp.int32, LANES)

    def row_step(r, carry):
        row_idx = jnp.full((LANES,), r, jnp.int32)

        def sum_step(i, c, row_idx=row_idx):
            acc, cols = c
            v = plsc.load_gather(rows_2d, [row_idx, cols])
            return acc + v, cols + LANES

        acc, _ = lax.fori_loop(
            0,
            VECS_PER_ROW,
            sum_step,
            (jnp.zeros((LANES,), jnp.float32), lane_iota),
        )
        total = jnp.sum(acc)
        total = total + jnp.float32(VOCAB_SIZE) * jnp.float32(PRIOR_VAL)
        ln_s = _ln_vec(jnp.full((LANES,), total))

        def log_step(i, cols, row_idx=row_idx, ln_s=ln_s):
            w = plsc.load_gather(rows_2d, [row_idx, cols])
            w = w + jnp.float32(PRIOR_VAL)
            plsc.store_scatter(rows_2d, [row_idx, cols], _ln_vec(w) - ln_s)
            return cols + LANES

        lax.fori_loop(0, VECS_PER_ROW, log_step, lane_iota)
        return carry

    def chunk_step(j, carry):
        off = pl.multiple_of(j * CHUNK, CHUNK)
        pltpu.async_copy(
            table_hbm.at[idx_v.at[pl.ds(off, CHUNK)]], rows_2d, sem
        ).wait()
        lax.fori_loop(0, CHUNK, row_step, jnp.int32(0))
        pltpu.sync_copy(rows_2d, out_hbm.at[pl.ds(base + off, CHUNK)])
        return carry

    lax.fori_loop(0, NUM_CHUNKS, chunk_step, jnp.int32(0))


@functools.partial(
    pl.kernel,
    out_type=jax.ShapeDtypeStruct((BATCH_SIZE, VOCAB_SIZE), jnp.float32),
    mesh=plsc.VectorSubcoreMesh(core_axis_name="c", subcore_axis_name="s"),
    scratch_types=[
        pltpu.VMEM((ROWS_PER_WORKER,), jnp.int32),
        pltpu.VMEM((CHUNK, VOCAB_SIZE), jnp.float32),
        pltpu.SemaphoreType.DMA,
    ],
    compiler_params=pltpu.CompilerParams(
        needs_layout_passes=False, use_tc_tiling_on_sc=False
    ),
)
def _bigram_gather_log(table_hbm, idx_hbm, out_hbm, idx_v, rows_2d, sem):
    _sc_body(table_hbm, idx_hbm, out_hbm, idx_v, rows_2d, sem)


def kernel(N, x):
    x = jnp.squeeze(x).astype(jnp.int32)
    return _bigram_gather_log(N.astype(jnp.float32), x)


# speedup vs baseline: 2.8067x; 2.8067x over previous
import functools

import jax
import jax.numpy as jnp
from jax import lax
from jax.experimental import pallas as pl
from jax.experimental.pallas import tpu as pltpu
from jax.experimental.pallas import tpu_sc as plsc

VOCAB_SIZE = 10000
BATCH_SIZE = 4096
PRIOR_VAL = 1.0

NUM_CORES = 2
NUM_SUBCORES = 16
LANES = 16
NUM_WORKERS = NUM_CORES * NUM_SUBCORES
ROWS_PER_WORKER = BATCH_SIZE // NUM_WORKERS
CHUNK = 8
NUM_CHUNKS = ROWS_PER_WORKER // CHUNK
VECS_PER_ROW = VOCAB_SIZE // LANES
UNROLL = 5

LN2 = 0.6931471805599453
SQRT2_HALF_BITS = 0x3F3504F3
EXP_OFFSET = 0x3F800000 - SQRT2_HALF_BITS
_C0 = -3.3329473846099855e-06
_C = (
    0.9999100019104868,
    -0.4993357263207794,
    0.33761055789635286,
    -0.271099350707996,
    0.1702861622184043,
)


def _ln_minus(w, c0):
    ix = plsc.bitcast(w, jnp.int32) + jnp.int32(EXP_OFFSET)
    e = (ix >> 23) - 127
    m = plsc.bitcast(
        (ix & jnp.int32(0x007FFFFF)) + jnp.int32(SQRT2_HALF_BITS), jnp.float32
    )
    t = m - jnp.float32(1.0)
    p = jnp.full((LANES,), jnp.float32(_C[4]))
    for c in (_C[3], _C[2], _C[1], _C[0]):
        p = p * t + jnp.float32(c)
    p = p * t + c0
    return e.astype(jnp.float32) * jnp.float32(LN2) + p


def _sc_body(table_hbm, idx_hbm, out_hbm, idx_v, rows_2d, sem):
    wid = lax.axis_index("s") * NUM_CORES + lax.axis_index("c")
    base = wid * ROWS_PER_WORKER
    pltpu.sync_copy(idx_hbm.at[pl.ds(base, ROWS_PER_WORKER)], idx_v)

    lane_iota = lax.iota(jnp.int32, LANES)
    zero_v = jnp.zeros((LANES,), jnp.float32)

    def row_step(r, carry):
        row_idx = jnp.full((LANES,), r, jnp.int32)

        @plsc.parallel_loop(0, VECS_PER_ROW, UNROLL, carry=(zero_v,) * UNROLL)
        def accs(i, accs_in):
            cols = lane_iota + i * LANES
            return tuple(
                accs_in[k]
                + plsc.load_gather(rows_2d, [row_idx, cols + (k * LANES)])
                for k in range(UNROLL)
            )

        acc = accs[0]
        for k in range(1, UNROLL):
            acc = acc + accs[k]
        total = jnp.sum(acc) + jnp.float32(VOCAB_SIZE * PRIOR_VAL)
        c0 = jnp.float32(_C0) - _ln_minus(
            jnp.full((LANES,), total), jnp.full((LANES,), jnp.float32(_C0))
        )

        @plsc.parallel_loop(0, VECS_PER_ROW, UNROLL)
        def _(i):
            cols = lane_iota + i * LANES
            for k in range(UNROLL):
                ck = cols + (k * LANES)
                w = plsc.load_gather(rows_2d, [row_idx, ck])
                w = w + jnp.float32(PRIOR_VAL)
                plsc.store_scatter(rows_2d, [row_idx, ck], _ln_minus(w, c0))

        return carry

    def chunk_step(j, carry):
        off = pl.multiple_of(j * CHUNK, CHUNK)
        pltpu.async_copy(
            table_hbm.at[idx_v.at[pl.ds(off, CHUNK)]], rows_2d, sem
        ).wait()
        lax.fori_loop(0, CHUNK, row_step, jnp.int32(0))
        pltpu.sync_copy(rows_2d, out_hbm.at[pl.ds(base + off, CHUNK)])
        return carry

    lax.fori_loop(0, NUM_CHUNKS, chunk_step, jnp.int32(0))


@functools.partial(
    pl.kernel,
    out_type=jax.ShapeDtypeStruct((BATCH_SIZE, VOCAB_SIZE), jnp.float32),
    mesh=plsc.VectorSubcoreMesh(core_axis_name="c", subcore_axis_name="s"),
    scratch_types=[
        pltpu.VMEM((ROWS_PER_WORKER,), jnp.int32),
        pltpu.VMEM((CHUNK, VOCAB_SIZE), jnp.float32),
        pltpu.SemaphoreType.DMA,
    ],
    compiler_params=pltpu.CompilerParams(
        needs_layout_passes=False, use_tc_tiling_on_sc=False
    ),
)
def _bigram_gather_log(table_hbm, idx_hbm, out_hbm, idx_v, rows_2d, sem):
    _sc_body(table_hbm, idx_hbm, out_hbm, idx_v, rows_2d, sem)


def kernel(N, x):
    x = jnp.squeeze(x).astype(jnp.int32)
    return _bigram_gather_log(N.astype(jnp.float32), x)


# speedup vs baseline: 3.9773x; 1.4171x over previous
import functools

import jax
import jax.numpy as jnp
from jax import lax
from jax.experimental import pallas as pl
from jax.experimental.pallas import tpu as pltpu
from jax.experimental.pallas import tpu_sc as plsc

VOCAB_SIZE = 10000
BATCH_SIZE = 4096
PRIOR_VAL = 1.0
PAD_D = 10240

NUM_CORES = 2
NUM_SUBCORES = 16
NUM_WORKERS = NUM_CORES * NUM_SUBCORES
ROWS_PER_WORKER = BATCH_SIZE // NUM_WORKERS
CHUNK = 8
NUM_CHUNKS = ROWS_PER_WORKER // CHUNK

ROWS_PER_BLOCK = 32


@functools.partial(
    pl.kernel,
    out_type=jax.ShapeDtypeStruct((BATCH_SIZE * PAD_D,), jnp.float32),
    mesh=plsc.VectorSubcoreMesh(core_axis_name="c", subcore_axis_name="s"),
    scratch_types=[
        pltpu.VMEM((ROWS_PER_WORKER,), jnp.int32),
        pltpu.VMEM((CHUNK, VOCAB_SIZE), jnp.float32),
        pltpu.SemaphoreType.DMA,
    ],
    compiler_params=pltpu.CompilerParams(use_tc_tiling_on_sc=False),
)
def _sc_gather(table_hbm, idx_hbm, g_hbm, idx_v, buf, sem):
    wid = lax.axis_index("s") * NUM_CORES + lax.axis_index("c")
    base = wid * ROWS_PER_WORKER
    pltpu.sync_copy(idx_hbm.at[pl.ds(base, ROWS_PER_WORKER)], idx_v)

    def chunk_step(j, carry):
        off = pl.multiple_of(j * CHUNK, CHUNK)
        pltpu.async_copy(
            table_hbm.at[idx_v.at[pl.ds(off, CHUNK)]], buf, sem
        ).wait()

        def row_step(r, c):
            grow = base + off + r
            pltpu.sync_copy(
                buf.at[r], g_hbm.at[pl.ds(grow * PAD_D, VOCAB_SIZE)]
            )
            return c

        lax.fori_loop(0, CHUNK, row_step, jnp.int32(0))
        return carry

    lax.fori_loop(0, NUM_CHUNKS, chunk_step, jnp.int32(0))


def _tc_log_body(g_ref, o_ref):
    x = g_ref[...].reshape(ROWS_PER_BLOCK, PAD_D)
    w = x + jnp.float32(PRIOR_VAL)
    mask = (
        lax.broadcasted_iota(jnp.int32, (ROWS_PER_BLOCK, PAD_D), 1)
        < VOCAB_SIZE
    )
    s = jnp.sum(jnp.where(mask, w, 0.0), axis=1, keepdims=True)
    o_ref[...] = (jnp.log(w) - jnp.log(s))[:, :VOCAB_SIZE]


def kernel(N, x):
    x = jnp.squeeze(x).astype(jnp.int32)
    g = _sc_gather(N.astype(jnp.float32), x)
    return pl.pallas_call(
        _tc_log_body,
        grid=(BATCH_SIZE // ROWS_PER_BLOCK,),
        in_specs=[
            pl.BlockSpec((ROWS_PER_BLOCK * PAD_D,), lambda i: (i,)),
        ],
        out_specs=pl.BlockSpec((ROWS_PER_BLOCK, VOCAB_SIZE), lambda i: (i, 0)),
        out_shape=jax.ShapeDtypeStruct((BATCH_SIZE, VOCAB_SIZE), jnp.float32),
    )(g)
